# baseline (speedup 1.0000x reference)
import functools

import jax
import jax.numpy as jnp
from jax import lax
from jax.experimental import pallas as pl
from jax.experimental.pallas import tpu as pltpu
from jax.experimental.pallas import tpu_sc as plsc

B, C, H, W = 64, 3, 512, 512
HW = H * W
NW = 32
P = 1024
ROWS_PER_CHUNK = P // W
NCH = HW // P
G = P // 16
RB = P + W + 16
D = 8


def _sc_warp(images_hbm, flows_hbm, out_hbm, tab_hbm,
             row_v, stage_b, stage_s, fl_v, w_v, idx_v, o_v,
             sbi0, sbi1, sbo0, sbo1, ssf0, ssf1, sg0, sg1, sso0, sso1):
    SBI = (sbi0, sbi1)
    SBO = (sbo0, sbo1)
    SSF = (ssf0, ssf1)
    SG = (sg0, sg1)
    SSO = (sso0, sso1)
    wid = lax.axis_index("s") * 2 + lax.axis_index("c")
    b0 = wid * 2
    b1 = b0 + 1

    def in_copies(b, k, p):
        r0 = k * ROWS_PER_CHUNK
        cps = []
        for c in range(3):
            src = pl.multiple_of((b * 3 + c) * HW + r0 * W, 8)
            cps.append(pltpu.make_async_copy(
                images_hbm.at[pl.ds(src, P)],
                row_v.at[p, c, pl.ds(0, P)], SBI[p]))
            nxt = jnp.minimum(r0 + ROWS_PER_CHUNK, H - 1) * W
            srcn = pl.multiple_of((b * 3 + c) * HW + nxt, 8)
            cps.append(pltpu.make_async_copy(
                images_hbm.at[pl.ds(srcn, W)],
                row_v.at[p, c, pl.ds(P, W)], SBI[p]))
        return cps

    def tab_out_copy(b, k, p):
        dst = pl.multiple_of(b * HW + k * P, 8)
        return pltpu.make_async_copy(
            stage_b.at[p], tab_hbm.at[pl.ds(dst, P), :], SBO[p])

    def interleave(p):
        @plsc.parallel_loop(0, G, unroll=8)
        def build_group(g):
            s = g * 16
            lane = lax.iota(jnp.int32, 16)
            rowidx = s + lane
            for c in range(3):
                va = row_v[p, c, pl.ds(s, 16)]
                vc = row_v[p, c, pl.ds(s + 1, 16)]
                vb = row_v[p, c, pl.ds(s + W, 16)]
                vd = row_v[p, c, pl.ds(s + W + 1, 16)]
                pac = plsc.bitcast(
                    plsc.pack(va, vc, format=plsc.PackFormat.INTERLEAVED),
                    jnp.int32)
                pbd = plsc.bitcast(
                    plsc.pack(vb, vd, format=plsc.PackFormat.INTERLEAVED),
                    jnp.int32)
                plsc.store_scatter(
                    stage_b.at[p],
                    [rowidx, jnp.full((16,), 2 * c, jnp.int32)], pac)
                plsc.store_scatter(
                    stage_b.at[p],
                    [rowidx, jnp.full((16,), 2 * c + 1, jnp.int32)], pbd)

    def build_pro(b):
        for cp in in_copies(b, 0, 0):
            cp.start()
        for cp in in_copies(b, 1, 1):
            cp.start()

    def build_step(b, j):
        for p in (0, 1):
            k = 2 * j + p
            for cp in in_copies(b, k, p):
                cp.wait()

            @pl.when(j >= 1)
            def _():
                tab_out_copy(b, k - 2, p).wait()

            interleave(p)
            tab_out_copy(b, k, p).start()

            @pl.when(j < NCH // 2 - 1)
            def _():
                for cp in in_copies(b, k + 2, p):
                    cp.start()

    def build_epi(b):
        tab_out_copy(b, NCH - 2, 0).wait()
        tab_out_copy(b, NCH - 1, 1).wait()

    def flow_copies(b, k, p):
        off = k * P
        fy = pl.multiple_of(2 * b * HW + off, 8)
        fx = pl.multiple_of((2 * b + 1) * HW + off, 8)
        return [
            pltpu.make_async_copy(flows_hbm.at[pl.ds(fy, P)],
                                  fl_v.at[p, 0], SSF[p]),
            pltpu.make_async_copy(flows_hbm.at[pl.ds(fx, P)],
                                  fl_v.at[p, 1], SSF[p]),
        ]

    def gather_copy(p):
        return pltpu.make_async_copy(
            tab_hbm.at[idx_v.at[p]], stage_s.at[p], SG[p])

    def out_copies(b, k, p):
        off = k * P
        cps = []
        for c in range(3):
            dst = pl.multiple_of((3 * b + c) * HW + off, 8)
            cps.append(pltpu.make_async_copy(
                o_v.at[p, c], out_hbm.at[pl.ds(dst, P)], SSO[p]))
        return cps

    def compute(b, k, p):
        off = k * P

        @plsc.parallel_loop(0, G, unroll=8)
        def compute_group(g):
            s = g * 16
            lane = lax.iota(jnp.int32, 16)
            hw = off + s + lane
            yi = hw >> 9
            xi = hw & 511
            fy = fl_v[p, 0, pl.ds(s, 16)]
            fx = fl_v[p, 1, pl.ds(s, 16)]
            sy = jnp.clip(yi.astype(jnp.float32) + fy, 0.0, float(H) - 1.0)
            sx = jnp.clip(xi.astype(jnp.float32) + fx, 0.0, float(W) - 1.0)
            y0 = jnp.minimum(sy.astype(jnp.int32), H - 2)
            x0 = jnp.minimum(sx.astype(jnp.int32), W - 2)
            ay = sy - y0.astype(jnp.float32)
            ax = sx - x0.astype(jnp.float32)
            by = 1.0 - ay
            bx = 1.0 - ax
            w_v[p, 0, pl.ds(s, 16)] = bx * by
            w_v[p, 1, pl.ds(s, 16)] = ax * by
            w_v[p, 2, pl.ds(s, 16)] = bx * ay
            w_v[p, 3, pl.ds(s, 16)] = ax * ay
            idx_v[p, pl.ds(s, 16)] = b * HW + y0 * W + x0

    def combine(p):
        @plsc.parallel_loop(0, G, unroll=8)
        def combine_group(g):
            s = g * 16
            lane = lax.iota(jnp.int32, 16)
            rowidx = s + lane
            wa = w_v[p, 0, pl.ds(s, 16)]
            wc = w_v[p, 1, pl.ds(s, 16)]
            wb = w_v[p, 2, pl.ds(s, 16)]
            wd = w_v[p, 3, pl.ds(s, 16)]
            for c in range(3):
                gac = plsc.load_gather(
                    stage_s.at[p],
                    [rowidx, jnp.full((16,), 2 * c, jnp.int32)])
                gbd = plsc.load_gather(
                    stage_s.at[p],
                    [rowidx, jnp.full((16,), 2 * c + 1, jnp.int32)])
                ta, tc = plsc.unpack(plsc.bitcast(gac, jnp.bfloat16),
                                     format=plsc.PackFormat.INTERLEAVED)
                tb, td = plsc.unpack(plsc.bitcast(gbd, jnp.bfloat16),
                                     format=plsc.PackFormat.INTERLEAVED)
                acc = wa * ta
                acc = acc + wc * tc
                acc = acc + wb * tb
                acc = acc + wd * td
                o_v[p, c, pl.ds(s, 16)] = acc

    def sample_pro(b):
        for cp in flow_copies(b, 0, 0):
            cp.start()

    def sample_step(b, j):
        for p in (0, 1):
            k = 2 * j + p
            for cp in flow_copies(b, k, p):
                cp.wait()
            compute(b, k, p)
            gather_copy(p).start()

            if p == 0:
                for cp in flow_copies(b, k + 1, 1):
                    cp.start()
            else:
                @pl.when(j < NCH // 2 - 1)
                def _():
                    for cp in flow_copies(b, k + 1, 0):
                        cp.start()

            q = 1 - p
            if p == 0:
                @pl.when(j >= 1)
                def _():
                    gather_copy(q).wait()

                    @pl.when(j >= 2)
                    def _():
                        for cp in out_copies(b, k - 3, q):
                            cp.wait()

                    combine(q)
                    for cp in out_copies(b, k - 1, q):
                        cp.start()
            else:
                gather_copy(q).wait()

                @pl.when(j >= 1)
                def _():
                    for cp in out_copies(b, k - 3, q):
                        cp.wait()

                combine(q)
                for cp in out_copies(b, k - 1, q):
                    cp.start()

    def sample_epi(b):
        gather_copy(1).wait()
        for cp in out_copies(b, NCH - 3, 1):
            cp.wait()
        combine(1)
        for cp in out_copies(b, NCH - 1, 1):
            cp.start()
        for cp in out_copies(b, NCH - 2, 0):
            cp.wait()
        for cp in out_copies(b, NCH - 1, 1):
            cp.wait()

    build_pro(b0)

    def phase_a(j, carry):
        build_step(b0, j)
        return carry

    lax.fori_loop(0, NCH // 2, phase_a, 0)
    build_epi(b0)

    build_pro(b1)
    sample_pro(b0)

    def phase_b(j, carry):
        build_step(b1, j)
        sample_step(b0, j)
        return carry

    lax.fori_loop(0, NCH // 2, phase_b, 0)
    build_epi(b1)
    sample_epi(b0)

    sample_pro(b1)

    def phase_c(j, carry):
        sample_step(b1, j)
        return carry

    lax.fori_loop(0, NCH // 2, phase_c, 0)
    sample_epi(b1)


@jax.jit
def kernel(images, flows):
    images_flat = images.reshape(B * C * HW)
    flows_flat = flows.reshape(B * 2 * HW)
    mesh = plsc.VectorSubcoreMesh(core_axis_name="c", subcore_axis_name="s")
    run = functools.partial(
        pl.kernel,
        mesh=mesh,
        out_type=(
            jax.ShapeDtypeStruct((B * C * HW,), jnp.float32),
            jax.ShapeDtypeStruct((B * HW, D), jnp.int32),
        ),
        scratch_types=[
            pltpu.VMEM((2, 3, RB), jnp.float32),
            pltpu.VMEM((2, P, D), jnp.int32),
            pltpu.VMEM((2, P, D), jnp.int32),
            pltpu.VMEM((2, 2, P), jnp.float32),
            pltpu.VMEM((2, 4, P), jnp.float32),
            pltpu.VMEM((2, P), jnp.int32),
            pltpu.VMEM((2, 3, P), jnp.float32),
        ] + [pltpu.SemaphoreType.DMA] * 10,
        compiler_params=pltpu.CompilerParams(
            use_tc_tiling_on_sc=False, needs_layout_passes=False),
    )(_sc_warp)
    out, _tab = run(images_flat, flows_flat)
    return out.reshape(B, C, H, W)


# speedup vs baseline: 1.0885x; 1.0885x over previous
import functools

import jax
import jax.numpy as jnp
from jax import lax
from jax.experimental import pallas as pl
from jax.experimental.pallas import tpu as pltpu
from jax.experimental.pallas import tpu_sc as plsc

B, C, H, W = 64, 3, 512, 512
HW = H * W
NW = 32
P = 1024
ROWS_PER_CHUNK = P // W
NCH = HW // P
G = P // 16
RB = P + W + 16
D = 8


def _sc_warp(images_hbm, flows_hbm, out_hbm, tab_hbm,
             row_v, stage_b, stage_s, fl_v, w_v, idx_v, o_v,
             sbi0, sbi1, sbo0, sbo1, ssf0, ssf1, sg0, sg1, sso0, sso1):
    SBI = (sbi0, sbi1)
    SBO = (sbo0, sbo1)
    SSF = (ssf0, ssf1)
    SG = (sg0, sg1)
    SSO = (sso0, sso1)
    wid = lax.axis_index("s") * 2 + lax.axis_index("c")
    b0 = wid * 2
    b1 = b0 + 1

    def in_copies(b, k, p):
        r0 = k * ROWS_PER_CHUNK
        cps = []
        for c in range(3):
            src = pl.multiple_of((b * 3 + c) * HW + r0 * W, 8)
            cps.append(pltpu.make_async_copy(
                images_hbm.at[pl.ds(src, P)],
                row_v.at[p, c, pl.ds(0, P)], SBI[p]))
            nxt = jnp.minimum(r0 + ROWS_PER_CHUNK, H - 1) * W
            srcn = pl.multiple_of((b * 3 + c) * HW + nxt, 8)
            cps.append(pltpu.make_async_copy(
                images_hbm.at[pl.ds(srcn, W)],
                row_v.at[p, c, pl.ds(P, W)], SBI[p]))
        return cps

    def tab_out_copy(b, k, p):
        dst = pl.multiple_of(b * HW + k * P, 8)
        return pltpu.make_async_copy(
            stage_b.at[p], tab_hbm.at[pl.ds(dst, P), :], SBO[p])

    def interleave(p):
        @plsc.parallel_loop(0, G, unroll=4)
        def build_group(g):
            s = g * 16
            lane = lax.iota(jnp.int32, 16)
            rowidx = s + lane
            for c in range(3):
                va = row_v[p, c, pl.ds(s, 16)]
                vc = row_v[p, c, pl.ds(s + 1, 16)]
                vb = row_v[p, c, pl.ds(s + W, 16)]
                vd = row_v[p, c, pl.ds(s + W + 1, 16)]
                pac = plsc.bitcast(
                    plsc.pack(va, vc, format=plsc.PackFormat.INTERLEAVED),
                    jnp.int32)
                pbd = plsc.bitcast(
                    plsc.pack(vb, vd, format=plsc.PackFormat.INTERLEAVED),
                    jnp.int32)
                plsc.store_scatter(
                    stage_b.at[p],
                    [rowidx, jnp.full((16,), 2 * c, jnp.int32)], pac)
                plsc.store_scatter(
                    stage_b.at[p],
                    [rowidx, jnp.full((16,), 2 * c + 1, jnp.int32)], pbd)

    def build_pro(b):
        for cp in in_copies(b, 0, 0):
            cp.start()
        for cp in in_copies(b, 1, 1):
            cp.start()

    def build_step(b, j):
        for p in (0, 1):
            k = 2 * j + p
            for cp in in_copies(b, k, p):
                cp.wait()

            @pl.when(j >= 1)
            def _():
                tab_out_copy(b, k - 2, p).wait()

            interleave(p)
            tab_out_copy(b, k, p).start()

            @pl.when(j < NCH // 2 - 1)
            def _():
                for cp in in_copies(b, k + 2, p):
                    cp.start()

    def build_epi(b):
        tab_out_copy(b, NCH - 2, 0).wait()
        tab_out_copy(b, NCH - 1, 1).wait()

    def flow_copies(b, k, p):
        off = k * P
        fy = pl.multiple_of(2 * b * HW + off, 8)
        fx = pl.multiple_of((2 * b + 1) * HW + off, 8)
        return [
            pltpu.make_async_copy(flows_hbm.at[pl.ds(fy, P)],
                                  fl_v.at[p, 0], SSF[p]),
            pltpu.make_async_copy(flows_hbm.at[pl.ds(fx, P)],
                                  fl_v.at[p, 1], SSF[p]),
        ]

    def gather_copy(p):
        return pltpu.make_async_copy(
            tab_hbm.at[idx_v.at[p]], stage_s.at[p], SG[p])

    def out_copies(b, k, p):
        off = k * P
        cps = []
        for c in range(3):
            dst = pl.multiple_of((3 * b + c) * HW + off, 8)
            cps.append(pltpu.make_async_copy(
                o_v.at[p, c], out_hbm.at[pl.ds(dst, P)], SSO[p]))
        return cps

    def compute(b, k, p):
        off = k * P

        @plsc.parallel_loop(0, G, unroll=4)
        def compute_group(g):
            s = g * 16
            lane = lax.iota(jnp.int32, 16)
            hw = off + s + lane
            yi = hw >> 9
            xi = hw & 511
            fy = fl_v[p, 0, pl.ds(s, 16)]
            fx = fl_v[p, 1, pl.ds(s, 16)]
            sy = jnp.clip(yi.astype(jnp.float32) + fy, 0.0, float(H) - 1.0)
            sx = jnp.clip(xi.astype(jnp.float32) + fx, 0.0, float(W) - 1.0)
            y0 = jnp.minimum(sy.astype(jnp.int32), H - 2)
            x0 = jnp.minimum(sx.astype(jnp.int32), W - 2)
            ay = sy - y0.astype(jnp.float32)
            ax = sx - x0.astype(jnp.float32)
            by = 1.0 - ay
            bx = 1.0 - ax
            w_v[p, 0, pl.ds(s, 16)] = bx * by
            w_v[p, 1, pl.ds(s, 16)] = ax * by
            w_v[p, 2, pl.ds(s, 16)] = bx * ay
            w_v[p, 3, pl.ds(s, 16)] = ax * ay
            idx_v[p, pl.ds(s, 16)] = b * HW + y0 * W + x0

    def combine(p):
        @plsc.parallel_loop(0, G, unroll=4)
        def combine_group(g):
            s = g * 16
            lane = lax.iota(jnp.int32, 16)
            rowidx = s + lane
            wa = w_v[p, 0, pl.ds(s, 16)]
            wc = w_v[p, 1, pl.ds(s, 16)]
            wb = w_v[p, 2, pl.ds(s, 16)]
            wd = w_v[p, 3, pl.ds(s, 16)]
            for c in range(3):
                gac = plsc.load_gather(
                    stage_s.at[p],
                    [rowidx, jnp.full((16,), 2 * c, jnp.int32)])
                gbd = plsc.load_gather(
                    stage_s.at[p],
                    [rowidx, jnp.full((16,), 2 * c + 1, jnp.int32)])
                ta, tc = plsc.unpack(plsc.bitcast(gac, jnp.bfloat16),
                                     format=plsc.PackFormat.INTERLEAVED)
                tb, td = plsc.unpack(plsc.bitcast(gbd, jnp.bfloat16),
                                     format=plsc.PackFormat.INTERLEAVED)
                acc = wa * ta
                acc = acc + wc * tc
                acc = acc + wb * tb
                acc = acc + wd * td
                o_v[p, c, pl.ds(s, 16)] = acc

    def sample_pro(b):
        for cp in flow_copies(b, 0, 0):
            cp.start()

    def sample_step(b, j):
        for p in (0, 1):
            k = 2 * j + p
            for cp in flow_copies(b, k, p):
                cp.wait()
            compute(b, k, p)
            gather_copy(p).start()

            if p == 0:
                for cp in flow_copies(b, k + 1, 1):
                    cp.start()
            else:
                @pl.when(j < NCH // 2 - 1)
                def _():
                    for cp in flow_copies(b, k + 1, 0):
                        cp.start()

            q = 1 - p
            if p == 0:
                @pl.when(j >= 1)
                def _():
                    gather_copy(q).wait()

                    @pl.when(j >= 2)
                    def _():
                        for cp in out_copies(b, k - 3, q):
                            cp.wait()

                    combine(q)
                    for cp in out_copies(b, k - 1, q):
                        cp.start()
            else:
                gather_copy(q).wait()

                @pl.when(j >= 1)
                def _():
                    for cp in out_copies(b, k - 3, q):
                        cp.wait()

                combine(q)
                for cp in out_copies(b, k - 1, q):
                    cp.start()

    def sample_epi(b):
        gather_copy(1).wait()
        for cp in out_copies(b, NCH - 3, 1):
            cp.wait()
        combine(1)
        for cp in out_copies(b, NCH - 1, 1):
            cp.start()
        for cp in out_copies(b, NCH - 2, 0):
            cp.wait()
        for cp in out_copies(b, NCH - 1, 1):
            cp.wait()

    build_pro(b0)

    def phase_a(j, carry):
        build_step(b0, j)
        return carry

    lax.fori_loop(0, NCH // 2, phase_a, 0)
    build_epi(b0)

    build_pro(b1)

    def phase_b1(j, carry):
        build_step(b1, j)
        return carry

    lax.fori_loop(0, NCH // 2, phase_b1, 0)
    build_epi(b1)

    sample_pro(b0)

    def phase_s0(j, carry):
        sample_step(b0, j)
        return carry

    lax.fori_loop(0, NCH // 2, phase_s0, 0)
    sample_epi(b0)

    sample_pro(b1)

    def phase_c(j, carry):
        sample_step(b1, j)
        return carry

    lax.fori_loop(0, NCH // 2, phase_c, 0)
    sample_epi(b1)


@jax.jit
def kernel(images, flows):
    images_flat = images.reshape(B * C * HW)
    flows_flat = flows.reshape(B * 2 * HW)
    mesh = plsc.VectorSubcoreMesh(core_axis_name="c", subcore_axis_name="s")
    run = functools.partial(
        pl.kernel,
        mesh=mesh,
        out_type=(
            jax.ShapeDtypeStruct((B * C * HW,), jnp.float32),
            jax.ShapeDtypeStruct((B * HW, D), jnp.int32),
        ),
        scratch_types=[
            pltpu.VMEM((2, 3, RB), jnp.float32),
            pltpu.VMEM((2, P, D), jnp.int32),
            pltpu.VMEM((2, P, D), jnp.int32),
            pltpu.VMEM((2, 2, P), jnp.float32),
            pltpu.VMEM((2, 4, P), jnp.float32),
            pltpu.VMEM((2, P), jnp.int32),
            pltpu.VMEM((2, 3, P), jnp.float32),
        ] + [pltpu.SemaphoreType.DMA] * 10,
        compiler_params=pltpu.CompilerParams(
            use_tc_tiling_on_sc=False, needs_layout_passes=False),
    )(_sc_warp)
    out, _tab = run(images_flat, flows_flat)
    return out.reshape(B, C, H, W)


# speedup vs baseline: 1.1397x; 1.0470x over previous
import functools

import jax
import jax.numpy as jnp
from jax import lax
from jax.experimental import pallas as pl
from jax.experimental.pallas import tpu as pltpu
from jax.experimental.pallas import tpu_sc as plsc

B, C, H, W = 64, 3, 512, 512
HW = H * W
NW = 32
P = 1024
ROWS_PER_CHUNK = P // W
NCH = HW // P
G = P // 16
RB = P + W + 16
D = 8


def _sc_warp(images_hbm, flows_hbm, out_hbm, tab_hbm,
             row_v, stage_b, stage_s, fl_v, w_v, idx_v, o_v,
             sbi0, sbi1, sbo0, sbo1, ssf0, ssf1, sg0, sg1, sso0, sso1):
    SBI = (sbi0, sbi1)
    SBO = (sbo0, sbo1)
    SSF = (ssf0, ssf1)
    SG = (sg0, sg1)
    SSO = (sso0, sso1)
    wid = lax.axis_index("s") * 2 + lax.axis_index("c")
    b0 = wid * 2
    b1 = b0 + 1

    def in_copies(b, k, p):
        r0 = k * ROWS_PER_CHUNK
        cps = []
        for c in range(3):
            src = pl.multiple_of((b * 3 + c) * HW + r0 * W, 8)
            cps.append(pltpu.make_async_copy(
                images_hbm.at[pl.ds(src, P)],
                row_v.at[p, c, pl.ds(0, P)], SBI[p]))
            nxt = jnp.minimum(r0 + ROWS_PER_CHUNK, H - 1) * W
            srcn = pl.multiple_of((b * 3 + c) * HW + nxt, 8)
            cps.append(pltpu.make_async_copy(
                images_hbm.at[pl.ds(srcn, W)],
                row_v.at[p, c, pl.ds(P, W)], SBI[p]))
        return cps

    def tab_out_copy(b, k, p):
        dst = pl.multiple_of(b * HW + k * P, 8)
        return pltpu.make_async_copy(
            stage_b.at[p], tab_hbm.at[pl.ds(dst, P), :], SBO[p])

    def interleave(p):
        @plsc.parallel_loop(0, G, unroll=4)
        def build_group(g):
            s = g * 16
            lane = lax.iota(jnp.int32, 16)
            rowidx = s + lane
            for c in range(3):
                va = row_v[p, c, pl.ds(s, 16)]
                vc = row_v[p, c, pl.ds(s + 1, 16)]
                vb = row_v[p, c, pl.ds(s + W, 16)]
                vd = row_v[p, c, pl.ds(s + W + 1, 16)]
                pac = plsc.bitcast(
                    plsc.pack(va, vc, format=plsc.PackFormat.INTERLEAVED),
                    jnp.int32)
                pbd = plsc.bitcast(
                    plsc.pack(vb, vd, format=plsc.PackFormat.INTERLEAVED),
                    jnp.int32)
                plsc.store_scatter(
                    stage_b.at[p],
                    [rowidx, jnp.full((16,), 2 * c, jnp.int32)], pac)
                plsc.store_scatter(
                    stage_b.at[p],
                    [rowidx, jnp.full((16,), 2 * c + 1, jnp.int32)], pbd)

    def build_pro(b):
        for cp in in_copies(b, 0, 0):
            cp.start()
        for cp in in_copies(b, 1, 1):
            cp.start()

    def build_step(b, j):
        for p in (0, 1):
            k = 2 * j + p
            for cp in in_copies(b, k, p):
                cp.wait()

            @pl.when(j >= 1)
            def _():
                tab_out_copy(b, k - 2, p).wait()

            interleave(p)
            tab_out_copy(b, k, p).start()

            @pl.when(j < NCH // 2 - 1)
            def _():
                for cp in in_copies(b, k + 2, p):
                    cp.start()

    def build_epi(b):
        tab_out_copy(b, NCH - 2, 0).wait()
        tab_out_copy(b, NCH - 1, 1).wait()

    def flow_copies(b, k, p):
        off = k * P
        fy = pl.multiple_of(2 * b * HW + off, 8)
        fx = pl.multiple_of((2 * b + 1) * HW + off, 8)
        return [
            pltpu.make_async_copy(flows_hbm.at[pl.ds(fy, P)],
                                  fl_v.at[p, 0], SSF[p]),
            pltpu.make_async_copy(flows_hbm.at[pl.ds(fx, P)],
                                  fl_v.at[p, 1], SSF[p]),
        ]

    def gather_copy(p):
        return pltpu.make_async_copy(
            tab_hbm.at[idx_v.at[p]], stage_s.at[p], SG[p])

    def out_copies(b, k, p):
        off = k * P
        cps = []
        for c in range(3):
            dst = pl.multiple_of((3 * b + c) * HW + off, 8)
            cps.append(pltpu.make_async_copy(
                o_v.at[p, c], out_hbm.at[pl.ds(dst, P)], SSO[p]))
        return cps

    def compute(b, k, p):
        off = k * P

        @plsc.parallel_loop(0, G, unroll=4)
        def compute_group(g):
            s = g * 16
            lane = lax.iota(jnp.int32, 16)
            hw = off + s + lane
            yi = hw >> 9
            xi = hw & 511
            fy = fl_v[p, 0, pl.ds(s, 16)]
            fx = fl_v[p, 1, pl.ds(s, 16)]
            sy = jnp.clip(yi.astype(jnp.float32) + fy, 0.0, float(H) - 1.0)
            sx = jnp.clip(xi.astype(jnp.float32) + fx, 0.0, float(W) - 1.0)
            y0 = jnp.minimum(sy.astype(jnp.int32), H - 2)
            x0 = jnp.minimum(sx.astype(jnp.int32), W - 2)
            ay = sy - y0.astype(jnp.float32)
            ax = sx - x0.astype(jnp.float32)
            by = 1.0 - ay
            bx = 1.0 - ax
            w_v[p, 0, pl.ds(s, 16)] = ax
            w_v[p, 1, pl.ds(s, 16)] = ay
            idx_v[p, pl.ds(s, 16)] = b * HW + y0 * W + x0

    def combine(p):
        @plsc.parallel_loop(0, G, unroll=4)
        def combine_group(g):
            s = g * 16
            lane = lax.iota(jnp.int32, 16)
            rowidx = s + lane
            ax = w_v[p, 0, pl.ds(s, 16)]
            ay = w_v[p, 1, pl.ds(s, 16)]
            bx = 1.0 - ax
            by = 1.0 - ay
            wa = bx * by
            wc = ax * by
            wb = bx * ay
            wd = ax * ay
            for c in range(3):
                gac = plsc.load_gather(
                    stage_s.at[p],
                    [rowidx, jnp.full((16,), 2 * c, jnp.int32)])
                gbd = plsc.load_gather(
                    stage_s.at[p],
                    [rowidx, jnp.full((16,), 2 * c + 1, jnp.int32)])
                ta, tc = plsc.unpack(plsc.bitcast(gac, jnp.bfloat16),
                                     format=plsc.PackFormat.INTERLEAVED)
                tb, td = plsc.unpack(plsc.bitcast(gbd, jnp.bfloat16),
                                     format=plsc.PackFormat.INTERLEAVED)
                acc = wa * ta
                acc = acc + wc * tc
                acc = acc + wb * tb
                acc = acc + wd * td
                o_v[p, c, pl.ds(s, 16)] = acc

    def sample_pro(b):
        for cp in flow_copies(b, 0, 0):
            cp.start()

    def sample_step(b, j):
        for p in (0, 1):
            k = 2 * j + p
            for cp in flow_copies(b, k, p):
                cp.wait()
            compute(b, k, p)
            gather_copy(p).start()

            if p == 0:
                for cp in flow_copies(b, k + 1, 1):
                    cp.start()
            else:
                @pl.when(j < NCH // 2 - 1)
                def _():
                    for cp in flow_copies(b, k + 1, 0):
                        cp.start()

            q = 1 - p
            if p == 0:
                @pl.when(j >= 1)
                def _():
                    gather_copy(q).wait()

                    @pl.when(j >= 2)
                    def _():
                        for cp in out_copies(b, k - 3, q):
                            cp.wait()

                    combine(q)
                    for cp in out_copies(b, k - 1, q):
                        cp.start()
            else:
                gather_copy(q).wait()

                @pl.when(j >= 1)
                def _():
                    for cp in out_copies(b, k - 3, q):
                        cp.wait()

                combine(q)
                for cp in out_copies(b, k - 1, q):
                    cp.start()

    def sample_epi(b):
        gather_copy(1).wait()
        for cp in out_copies(b, NCH - 3, 1):
            cp.wait()
        combine(1)
        for cp in out_copies(b, NCH - 1, 1):
            cp.start()
        for cp in out_copies(b, NCH - 2, 0):
            cp.wait()
        for cp in out_copies(b, NCH - 1, 1):
            cp.wait()

    build_pro(b0)

    def phase_a(j, carry):
        build_step(b0, j)
        return carry

    lax.fori_loop(0, NCH // 2, phase_a, 0)
    build_epi(b0)

    build_pro(b1)
    sample_pro(b0)

    def phase_b(j, carry):
        build_step(b1, j)
        sample_step(b0, j)
        return carry

    lax.fori_loop(0, NCH // 2, phase_b, 0)
    build_epi(b1)
    sample_epi(b0)

    sample_pro(b1)

    def phase_c(j, carry):
        sample_step(b1, j)
        return carry

    lax.fori_loop(0, NCH // 2, phase_c, 0)
    sample_epi(b1)


@jax.jit
def kernel(images, flows):
    images_flat = images.reshape(B * C * HW)
    flows_flat = flows.reshape(B * 2 * HW)
    mesh = plsc.VectorSubcoreMesh(core_axis_name="c", subcore_axis_name="s")
    run = functools.partial(
        pl.kernel,
        mesh=mesh,
        out_type=(
            jax.ShapeDtypeStruct((B * C * HW,), jnp.float32),
            jax.ShapeDtypeStruct((B * HW, D), jnp.int32),
        ),
        scratch_types=[
            pltpu.VMEM((2, 3, RB), jnp.float32),
            pltpu.VMEM((2, P, D), jnp.int32),
            pltpu.VMEM((2, P, D), jnp.int32),
            pltpu.VMEM((2, 2, P), jnp.float32),
            pltpu.VMEM((2, 2, P), jnp.float32),
            pltpu.VMEM((2, P), jnp.int32),
            pltpu.VMEM((2, 3, P), jnp.float32),
        ] + [pltpu.SemaphoreType.DMA] * 10,
        compiler_params=pltpu.CompilerParams(
            use_tc_tiling_on_sc=False, needs_layout_passes=False),
    )(_sc_warp)
    out, _tab = run(images_flat, flows_flat)
    return out.reshape(B, C, H, W)


# speedup vs baseline: 1.2610x; 1.1064x over previous
import functools

import jax
import jax.numpy as jnp
from jax import lax
from jax.experimental import pallas as pl
from jax.experimental.pallas import tpu as pltpu
from jax.experimental.pallas import tpu_sc as plsc

B, C, H, W = 64, 3, 512, 512
HW = H * W
NW = 32
P = 2048
ROWS_PER_CHUNK = P // W
NCH = HW // P
G = P // 16
RB = P + W + 16
D = 8


def _sc_warp(images_hbm, flows_hbm, out_hbm, tab_hbm,
             row_v, stage_b, stage_s, fl_v, w_v, idx_v, o_v,
             sbi0, sbi1, sbo0, sbo1, ssf0, ssf1, sg0, sg1, sso0, sso1):
    SBI = (sbi0, sbi1)
    SBO = (sbo0, sbo1)
    SSF = (ssf0, ssf1)
    SG = (sg0, sg1)
    SSO = (sso0, sso1)
    wid = lax.axis_index("s") * 2 + lax.axis_index("c")
    b0 = wid * 2
    b1 = b0 + 1

    def in_copies(b, k, p):
        r0 = k * ROWS_PER_CHUNK
        cps = []
        for c in range(3):
            src = pl.multiple_of((b * 3 + c) * HW + r0 * W, 8)
            cps.append(pltpu.make_async_copy(
                images_hbm.at[pl.ds(src, P)],
                row_v.at[p, c, pl.ds(0, P)], SBI[p]))
            nxt = jnp.minimum(r0 + ROWS_PER_CHUNK, H - 1) * W
            srcn = pl.multiple_of((b * 3 + c) * HW + nxt, 8)
            cps.append(pltpu.make_async_copy(
                images_hbm.at[pl.ds(srcn, W)],
                row_v.at[p, c, pl.ds(P, W)], SBI[p]))
        return cps

    def tab_out_copy(b, k, p):
        dst = pl.multiple_of(b * HW + k * P, 8)
        return pltpu.make_async_copy(
            stage_b.at[p], tab_hbm.at[pl.ds(dst, P), :], SBO[p])

    def interleave(p):
        @plsc.parallel_loop(0, G, unroll=4)
        def build_group(g):
            s = g * 16
            lane = lax.iota(jnp.int32, 16)
            rowidx = s + lane
            for c in range(3):
                va = row_v[p, c, pl.ds(s, 16)]
                vc = row_v[p, c, pl.ds(s + 1, 16)]
                vb = row_v[p, c, pl.ds(s + W, 16)]
                vd = row_v[p, c, pl.ds(s + W + 1, 16)]
                pac = plsc.bitcast(
                    plsc.pack(va, vc, format=plsc.PackFormat.INTERLEAVED),
                    jnp.int32)
                pbd = plsc.bitcast(
                    plsc.pack(vb, vd, format=plsc.PackFormat.INTERLEAVED),
                    jnp.int32)
                plsc.store_scatter(
                    stage_b.at[p],
                    [rowidx, jnp.full((16,), 2 * c, jnp.int32)], pac)
                plsc.store_scatter(
                    stage_b.at[p],
                    [rowidx, jnp.full((16,), 2 * c + 1, jnp.int32)], pbd)

    def build_pro(b):
        for cp in in_copies(b, 0, 0):
            cp.start()
        for cp in in_copies(b, 1, 1):
            cp.start()

    def build_step(b, j):
        for p in (0, 1):
            k = 2 * j + p
            for cp in in_copies(b, k, p):
                cp.wait()

            @pl.when(j >= 1)
            def _():
                tab_out_copy(b, k - 2, p).wait()

            interleave(p)
            tab_out_copy(b, k, p).start()

            @pl.when(j < NCH // 2 - 1)
            def _():
                for cp in in_copies(b, k + 2, p):
                    cp.start()

    def build_epi(b):
        tab_out_copy(b, NCH - 2, 0).wait()
        tab_out_copy(b, NCH - 1, 1).wait()

    def flow_copies(b, k, p):
        off = k * P
        fy = pl.multiple_of(2 * b * HW + off, 8)
        fx = pl.multiple_of((2 * b + 1) * HW + off, 8)
        return [
            pltpu.make_async_copy(flows_hbm.at[pl.ds(fy, P)],
                                  fl_v.at[p, 0], SSF[p]),
            pltpu.make_async_copy(flows_hbm.at[pl.ds(fx, P)],
                                  fl_v.at[p, 1], SSF[p]),
        ]

    def gather_copy(p):
        return pltpu.make_async_copy(
            tab_hbm.at[idx_v.at[p]], stage_s.at[p], SG[p])

    def out_copies(b, k, p):
        off = k * P
        cps = []
        for c in range(3):
            dst = pl.multiple_of((3 * b + c) * HW + off, 8)
            cps.append(pltpu.make_async_copy(
                o_v.at[p, c], out_hbm.at[pl.ds(dst, P)], SSO[p]))
        return cps

    def compute(b, k, p):
        off = k * P

        @plsc.parallel_loop(0, G, unroll=4)
        def compute_group(g):
            s = g * 16
            lane = lax.iota(jnp.int32, 16)
            hw = off + s + lane
            yi = hw >> 9
            xi = hw & 511
            fy = fl_v[p, 0, pl.ds(s, 16)]
            fx = fl_v[p, 1, pl.ds(s, 16)]
            sy = jnp.clip(yi.astype(jnp.float32) + fy, 0.0, float(H) - 1.0)
            sx = jnp.clip(xi.astype(jnp.float32) + fx, 0.0, float(W) - 1.0)
            y0 = jnp.minimum(sy.astype(jnp.int32), H - 2)
            x0 = jnp.minimum(sx.astype(jnp.int32), W - 2)
            ay = sy - y0.astype(jnp.float32)
            ax = sx - x0.astype(jnp.float32)
            by = 1.0 - ay
            bx = 1.0 - ax
            w_v[p, 0, pl.ds(s, 16)] = ax
            w_v[p, 1, pl.ds(s, 16)] = ay
            idx_v[p, pl.ds(s, 16)] = b * HW + y0 * W + x0

    def combine(p):
        @plsc.parallel_loop(0, G, unroll=4)
        def combine_group(g):
            s = g * 16
            lane = lax.iota(jnp.int32, 16)
            rowidx = s + lane
            ax = w_v[p, 0, pl.ds(s, 16)]
            ay = w_v[p, 1, pl.ds(s, 16)]
            bx = 1.0 - ax
            by = 1.0 - ay
            wa = bx * by
            wc = ax * by
            wb = bx * ay
            wd = ax * ay
            for c in range(3):
                gac = plsc.load_gather(
                    stage_s.at[p],
                    [rowidx, jnp.full((16,), 2 * c, jnp.int32)])
                gbd = plsc.load_gather(
                    stage_s.at[p],
                    [rowidx, jnp.full((16,), 2 * c + 1, jnp.int32)])
                ta, tc = plsc.unpack(plsc.bitcast(gac, jnp.bfloat16),
                                     format=plsc.PackFormat.INTERLEAVED)
                tb, td = plsc.unpack(plsc.bitcast(gbd, jnp.bfloat16),
                                     format=plsc.PackFormat.INTERLEAVED)
                acc = wa * ta
                acc = acc + wc * tc
                acc = acc + wb * tb
                acc = acc + wd * td
                o_v[p, c, pl.ds(s, 16)] = acc

    def sample_pro(b):
        for cp in flow_copies(b, 0, 0):
            cp.start()

    def sample_step(b, j):
        for p in (0, 1):
            k = 2 * j + p
            for cp in flow_copies(b, k, p):
                cp.wait()
            compute(b, k, p)
            gather_copy(p).start()

            if p == 0:
                for cp in flow_copies(b, k + 1, 1):
                    cp.start()
            else:
                @pl.when(j < NCH // 2 - 1)
                def _():
                    for cp in flow_copies(b, k + 1, 0):
                        cp.start()

            q = 1 - p
            if p == 0:
                @pl.when(j >= 1)
                def _():
                    gather_copy(q).wait()

                    @pl.when(j >= 2)
                    def _():
                        for cp in out_copies(b, k - 3, q):
                            cp.wait()

                    combine(q)
                    for cp in out_copies(b, k - 1, q):
                        cp.start()
            else:
                gather_copy(q).wait()

                @pl.when(j >= 1)
                def _():
                    for cp in out_copies(b, k - 3, q):
                        cp.wait()

                combine(q)
                for cp in out_copies(b, k - 1, q):
                    cp.start()

    def sample_epi(b):
        gather_copy(1).wait()
        for cp in out_copies(b, NCH - 3, 1):
            cp.wait()
        combine(1)
        for cp in out_copies(b, NCH - 1, 1):
            cp.start()
        for cp in out_copies(b, NCH - 2, 0):
            cp.wait()
        for cp in out_copies(b, NCH - 1, 1):
            cp.wait()

    build_pro(b0)

    def phase_a(j, carry):
        build_step(b0, j)
        return carry

    lax.fori_loop(0, NCH // 2, phase_a, 0)
    build_epi(b0)

    build_pro(b1)
    sample_pro(b0)

    def phase_b(j, carry):
        build_step(b1, j)
        sample_step(b0, j)
        return carry

    lax.fori_loop(0, NCH // 2, phase_b, 0)
    build_epi(b1)
    sample_epi(b0)

    sample_pro(b1)

    def phase_c(j, carry):
        sample_step(b1, j)
        return carry

    lax.fori_loop(0, NCH // 2, phase_c, 0)
    sample_epi(b1)


@jax.jit
def kernel(images, flows):
    images_flat = images.reshape(B * C * HW)
    flows_flat = flows.reshape(B * 2 * HW)
    mesh = plsc.VectorSubcoreMesh(core_axis_name="c", subcore_axis_name="s")
    run = functools.partial(
        pl.kernel,
        mesh=mesh,
        out_type=(
            jax.ShapeDtypeStruct((B * C * HW,), jnp.float32),
            jax.ShapeDtypeStruct((B * HW, D), jnp.int32),
        ),
        scratch_types=[
            pltpu.VMEM((2, 3, RB), jnp.float32),
            pltpu.VMEM((2, P, D), jnp.int32),
            pltpu.VMEM((2, P, D), jnp.int32),
            pltpu.VMEM((2, 2, P), jnp.float32),
            pltpu.VMEM((2, 2, P), jnp.float32),
            pltpu.VMEM((2, P), jnp.int32),
            pltpu.VMEM((2, 3, P), jnp.float32),
        ] + [pltpu.SemaphoreType.DMA] * 10,
        compiler_params=pltpu.CompilerParams(
            use_tc_tiling_on_sc=False, needs_layout_passes=False),
    )(_sc_warp)
    out, _tab = run(images_flat, flows_flat)
    return out.reshape(B, C, H, W)


# speedup vs baseline: 1.2622x; 1.0009x over previous
import functools

import jax
import jax.numpy as jnp
from jax import lax
from jax.experimental import pallas as pl
from jax.experimental.pallas import tpu as pltpu
from jax.experimental.pallas import tpu_sc as plsc

B, C, H, W = 64, 3, 512, 512
HW = H * W
NW = 32
P = 2048
ROWS_PER_CHUNK = P // W
NCH = HW // P
G = P // 16
RB = P + W + 16
D = 8


def _sc_warp(images_hbm, flows_hbm, out_hbm, tab_hbm,
             row_v, stage_b, stage_s, fl_v, w_v, idx_v, o_v,
             sbi0, sbi1, sbo0, sbo1, ssf0, ssf1, sg0, sg1, sso0, sso1):
    SBI = (sbi0, sbi1)
    SBO = (sbo0, sbo1)
    SSF = (ssf0, ssf1)
    SG = (sg0, sg1)
    SSO = (sso0, sso1)
    wid = lax.axis_index("s") * 2 + lax.axis_index("c")
    b0 = wid * 2
    b1 = b0 + 1

    def issue_in(b, k, p):
        for c in range(3):
            base = (b * 3 + c) * HW

            @pl.when(k < NCH - 1)
            def _():
                src = pl.multiple_of(base + k * P, 8)
                pltpu.make_async_copy(
                    images_hbm.at[pl.ds(src, P + W)],
                    row_v.at[p, c, pl.ds(0, P + W)], SBI[p]).start()

            @pl.when(k == NCH - 1)
            def _():
                src = pl.multiple_of(base + k * P, 8)
                pltpu.make_async_copy(
                    images_hbm.at[pl.ds(src, P)],
                    row_v.at[p, c, pl.ds(0, P)], SBI[p]).start()
                srcn = pl.multiple_of(base + (H - 1) * W, 8)
                pltpu.make_async_copy(
                    images_hbm.at[pl.ds(srcn, W)],
                    row_v.at[p, c, pl.ds(P, W)], SBI[p]).start()

    def wait_in(b, k, p):
        for c in range(3):
            pltpu.make_async_copy(
                images_hbm.at[pl.ds(0, P + W)],
                row_v.at[p, c, pl.ds(0, P + W)], SBI[p]).wait()

    def tab_out_copy(b, k, p):
        dst = pl.multiple_of(b * HW + k * P, 8)
        return pltpu.make_async_copy(
            stage_b.at[p], tab_hbm.at[pl.ds(dst, P), :], SBO[p])

    def interleave(p):
        @plsc.parallel_loop(0, G, unroll=4)
        def build_group(g):
            s = g * 16
            lane = lax.iota(jnp.int32, 16)
            rowidx = s + lane
            for c in range(3):
                va = row_v[p, c, pl.ds(s, 16)]
                vc = row_v[p, c, pl.ds(s + 1, 16)]
                vb = row_v[p, c, pl.ds(s + W, 16)]
                vd = row_v[p, c, pl.ds(s + W + 1, 16)]
                pac = plsc.bitcast(
                    plsc.pack(va, vc, format=plsc.PackFormat.INTERLEAVED),
                    jnp.int32)
                pbd = plsc.bitcast(
                    plsc.pack(vb, vd, format=plsc.PackFormat.INTERLEAVED),
                    jnp.int32)
                plsc.store_scatter(
                    stage_b.at[p],
                    [rowidx, jnp.full((16,), 2 * c, jnp.int32)], pac)
                plsc.store_scatter(
                    stage_b.at[p],
                    [rowidx, jnp.full((16,), 2 * c + 1, jnp.int32)], pbd)

    def build_pro(b):
        issue_in(b, 0, 0)
        issue_in(b, 1, 1)

    def build_step(b, j):
        for p in (0, 1):
            k = 2 * j + p
            wait_in(b, k, p)

            @pl.when(j >= 1)
            def _():
                tab_out_copy(b, k - 2, p).wait()

            interleave(p)
            tab_out_copy(b, k, p).start()

            @pl.when(j < NCH // 2 - 1)
            def _():
                issue_in(b, k + 2, p)

    def build_epi(b):
        tab_out_copy(b, NCH - 2, 0).wait()
        tab_out_copy(b, NCH - 1, 1).wait()

    def flow_copies(b, k, p):
        off = k * P
        fy = pl.multiple_of(2 * b * HW + off, 8)
        fx = pl.multiple_of((2 * b + 1) * HW + off, 8)
        return [
            pltpu.make_async_copy(flows_hbm.at[pl.ds(fy, P)],
                                  fl_v.at[p, 0], SSF[p]),
            pltpu.make_async_copy(flows_hbm.at[pl.ds(fx, P)],
                                  fl_v.at[p, 1], SSF[p]),
        ]

    def gather_copy(p):
        return pltpu.make_async_copy(
            tab_hbm.at[idx_v.at[p]], stage_s.at[p], SG[p])

    def out_copies(b, k, p):
        off = k * P
        cps = []
        for c in range(3):
            dst = pl.multiple_of((3 * b + c) * HW + off, 8)
            cps.append(pltpu.make_async_copy(
                o_v.at[p, c], out_hbm.at[pl.ds(dst, P)], SSO[p]))
        return cps

    def compute(b, k, p):
        off = k * P

        @plsc.parallel_loop(0, G, unroll=4)
        def compute_group(g):
            s = g * 16
            lane = lax.iota(jnp.int32, 16)
            hw = off + s + lane
            yi = hw >> 9
            xi = hw & 511
            fy = fl_v[p, 0, pl.ds(s, 16)]
            fx = fl_v[p, 1, pl.ds(s, 16)]
            sy = jnp.clip(yi.astype(jnp.float32) + fy, 0.0, float(H) - 1.0)
            sx = jnp.clip(xi.astype(jnp.float32) + fx, 0.0, float(W) - 1.0)
            y0 = jnp.minimum(sy.astype(jnp.int32), H - 2)
            x0 = jnp.minimum(sx.astype(jnp.int32), W - 2)
            ay = sy - y0.astype(jnp.float32)
            ax = sx - x0.astype(jnp.float32)
            by = 1.0 - ay
            bx = 1.0 - ax
            w_v[p, 0, pl.ds(s, 16)] = ax
            w_v[p, 1, pl.ds(s, 16)] = ay
            idx_v[p, pl.ds(s, 16)] = b * HW + y0 * W + x0

    def combine(p):
        @plsc.parallel_loop(0, G, unroll=4)
        def combine_group(g):
            s = g * 16
            lane = lax.iota(jnp.int32, 16)
            rowidx = s + lane
            ax = w_v[p, 0, pl.ds(s, 16)]
            ay = w_v[p, 1, pl.ds(s, 16)]
            bx = 1.0 - ax
            by = 1.0 - ay
            wa = bx * by
            wc = ax * by
            wb = bx * ay
            wd = ax * ay
            for c in range(3):
                gac = plsc.load_gather(
                    stage_s.at[p],
                    [rowidx, jnp.full((16,), 2 * c, jnp.int32)])
                gbd = plsc.load_gather(
                    stage_s.at[p],
                    [rowidx, jnp.full((16,), 2 * c + 1, jnp.int32)])
                ta, tc = plsc.unpack(plsc.bitcast(gac, jnp.bfloat16),
                                     format=plsc.PackFormat.INTERLEAVED)
                tb, td = plsc.unpack(plsc.bitcast(gbd, jnp.bfloat16),
                                     format=plsc.PackFormat.INTERLEAVED)
                acc = wa * ta
                acc = acc + wc * tc
                acc = acc + wb * tb
                acc = acc + wd * td
                o_v[p, c, pl.ds(s, 16)] = acc

    def sample_pro(b):
        for cp in flow_copies(b, 0, 0):
            cp.start()

    def sample_step(b, j):
        for p in (0, 1):
            k = 2 * j + p
            for cp in flow_copies(b, k, p):
                cp.wait()
            compute(b, k, p)
            gather_copy(p).start()

            if p == 0:
                for cp in flow_copies(b, k + 1, 1):
                    cp.start()
            else:
                @pl.when(j < NCH // 2 - 1)
                def _():
                    for cp in flow_copies(b, k + 1, 0):
                        cp.start()

            q = 1 - p
            if p == 0:
                @pl.when(j >= 1)
                def _():
                    gather_copy(q).wait()

                    @pl.when(j >= 2)
                    def _():
                        for cp in out_copies(b, k - 3, q):
                            cp.wait()

                    combine(q)
                    for cp in out_copies(b, k - 1, q):
                        cp.start()
            else:
                gather_copy(q).wait()

                @pl.when(j >= 1)
                def _():
                    for cp in out_copies(b, k - 3, q):
                        cp.wait()

                combine(q)
                for cp in out_copies(b, k - 1, q):
                    cp.start()

    def sample_epi(b):
        gather_copy(1).wait()
        for cp in out_copies(b, NCH - 3, 1):
            cp.wait()
        combine(1)
        for cp in out_copies(b, NCH - 1, 1):
            cp.start()
        for cp in out_copies(b, NCH - 2, 0):
            cp.wait()
        for cp in out_copies(b, NCH - 1, 1):
            cp.wait()

    build_pro(b0)

    def phase_a(j, carry):
        build_step(b0, j)
        return carry

    lax.fori_loop(0, NCH // 2, phase_a, 0)
    build_epi(b0)

    build_pro(b1)
    sample_pro(b0)

    def phase_b(j, carry):
        build_step(b1, j)
        sample_step(b0, j)
        return carry

    lax.fori_loop(0, NCH // 2, phase_b, 0)
    build_epi(b1)
    sample_epi(b0)

    sample_pro(b1)

    def phase_c(j, carry):
        sample_step(b1, j)
        return carry

    lax.fori_loop(0, NCH // 2, phase_c, 0)
    sample_epi(b1)


@jax.jit
def kernel(images, flows):
    images_flat = images.reshape(B * C * HW)
    flows_flat = flows.reshape(B * 2 * HW)
    mesh = plsc.VectorSubcoreMesh(core_axis_name="c", subcore_axis_name="s")
    run = functools.partial(
        pl.kernel,
        mesh=mesh,
        out_type=(
            jax.ShapeDtypeStruct((B * C * HW,), jnp.float32),
            jax.ShapeDtypeStruct((B * HW, D), jnp.int32),
        ),
        scratch_types=[
            pltpu.VMEM((2, 3, RB), jnp.float32),
            pltpu.VMEM((2, P, D), jnp.int32),
            pltpu.VMEM((2, P, D), jnp.int32),
            pltpu.VMEM((2, 2, P), jnp.float32),
            pltpu.VMEM((2, 2, P), jnp.float32),
            pltpu.VMEM((2, P), jnp.int32),
            pltpu.VMEM((2, 3, P), jnp.float32),
        ] + [pltpu.SemaphoreType.DMA] * 10,
        compiler_params=pltpu.CompilerParams(
            use_tc_tiling_on_sc=False, needs_layout_passes=False),
    )(_sc_warp)
    out, _tab = run(images_flat, flows_flat)
    return out.reshape(B, C, H, W)
